# Initial kernel scaffold; baseline (speedup 1.0000x reference)
import jax
import jax.numpy as jnp
from jax.experimental import pallas as pl


def kernel(x, edge_index, W):
    raise NotImplementedError("write your pallas kernel here")


# speedup vs baseline: 2.9617x; 2.9617x over previous
import functools

import jax
import jax.numpy as jnp
from jax import lax
from jax.experimental import pallas as pl
from jax.experimental.pallas import tpu as pltpu
from jax.experimental.pallas import tpu_sc as plsc

N_NODES = 10000
N_EDGES = 320000
D_FEAT = 128
N_OUT = 4

NC, NS, L = 2, 16, 16
NW = NC * NS
EPW = N_EDGES // NW
GROUPS = EPW // L


def _mm_body(x_ref, wt_ref, y_ref):
    y_ref[...] = jnp.dot(x_ref[...], wt_ref[...],
                         preferred_element_type=jnp.float32)


def _node_proj(x, wt):
    return pl.pallas_call(
        _mm_body,
        out_shape=jax.ShapeDtypeStruct((N_NODES, N_OUT), jnp.float32),
    )(x, wt)


@functools.cache
def _make_edge_kernel():
    mesh = plsc.VectorSubcoreMesh(core_axis_name="c", subcore_axis_name="s")

    @functools.partial(
        pl.kernel,
        mesh=mesh,
        out_type=jax.ShapeDtypeStruct((N_EDGES * N_OUT,), jnp.float32),
        scratch_types=[
            pltpu.VMEM((N_NODES * N_OUT,), jnp.float32),
            pltpu.VMEM((EPW,), jnp.int32),
            pltpu.VMEM((EPW,), jnp.int32),
            pltpu.VMEM((EPW * N_OUT,), jnp.float32),
        ],
        compiler_params=pltpu.CompilerParams(needs_layout_passes=False),
    )
    def edge_kernel(y_hbm, row_hbm, col_hbm, out_hbm,
                    y_v, rows_v, cols_v, out_v):
        wid = lax.axis_index("s") * NC + lax.axis_index("c")
        base = wid * EPW
        pltpu.sync_copy(y_hbm, y_v)
        pltpu.sync_copy(row_hbm.at[pl.ds(base, EPW)], rows_v)
        pltpu.sync_copy(col_hbm.at[pl.ds(base, EPW)], cols_v)

        lane = lax.iota(jnp.int32, L)

        def body(g, carry):
            rv = rows_v[pl.ds(g * L, L)] * N_OUT
            cv = cols_v[pl.ds(g * L, L)] * N_OUT
            obase = g * (L * N_OUT) + lane * N_OUT
            for j in range(N_OUT):
                a = plsc.load_gather(y_v, [rv + j])
                b = plsc.load_gather(y_v, [cv + j])
                e2 = jnp.exp((a + b) * 2.0)
                t = 1.0 - 2.0 / (e2 + 1.0)
                plsc.store_scatter(out_v, [obase + j], t)
            return carry

        lax.fori_loop(0, GROUPS, body, 0)
        pltpu.sync_copy(out_v, out_hbm.at[pl.ds(base * N_OUT, EPW * N_OUT)])

    return edge_kernel


def kernel(x, edge_index, W):
    y = _node_proj(x, W.T)
    out_flat = _make_edge_kernel()(y.reshape(-1), edge_index[0], edge_index[1])
    return out_flat.reshape(N_EDGES, 2, 2)


# speedup vs baseline: 27.9990x; 9.4538x over previous
import functools

import jax
import jax.numpy as jnp
from jax import lax
from jax.experimental import pallas as pl
from jax.experimental.pallas import tpu as pltpu
from jax.experimental.pallas import tpu_sc as plsc

N_NODES = 10000
N_EDGES = 320000
D_FEAT = 128
N_OUT = 4

NC, NS, L = 2, 16, 16
NW = NC * NS
EPW = N_EDGES // NW
GROUPS = EPW // L


def _mm_body(x_ref, wt_ref, y_ref):
    y_ref[...] = jnp.dot(x_ref[...], wt_ref[...],
                         preferred_element_type=jnp.float32)


def _node_proj(x, wt):
    return pl.pallas_call(
        _mm_body,
        out_shape=jax.ShapeDtypeStruct((N_NODES, N_OUT), jnp.float32),
    )(x, wt)


@functools.cache
def _make_edge_kernel():
    mesh = plsc.VectorSubcoreMesh(core_axis_name="c", subcore_axis_name="s")

    @functools.partial(
        pl.kernel,
        mesh=mesh,
        out_type=jax.ShapeDtypeStruct((N_OUT, N_EDGES), jnp.float32),
        scratch_types=[
            pltpu.VMEM((N_NODES * N_OUT,), jnp.float32),
            pltpu.VMEM((EPW,), jnp.int32),
            pltpu.VMEM((EPW,), jnp.int32),
        ] + [pltpu.VMEM((EPW,), jnp.float32) for _ in range(N_OUT)],
        compiler_params=pltpu.CompilerParams(
            needs_layout_passes=False, use_tc_tiling_on_sc=False),
    )
    def edge_kernel(y_hbm, row_hbm, col_hbm, out_hbm,
                    y_v, rows_v, cols_v, o0, o1, o2, o3):
        outs = (o0, o1, o2, o3)
        wid = lax.axis_index("s") * NC + lax.axis_index("c")
        base = wid * EPW
        pltpu.sync_copy(y_hbm, y_v)
        pltpu.sync_copy(row_hbm.at[pl.ds(base, EPW)], rows_v)
        pltpu.sync_copy(col_hbm.at[pl.ds(base, EPW)], cols_v)

        def body(g, carry):
            rv = rows_v[pl.ds(g * L, L)] * N_OUT
            cv = cols_v[pl.ds(g * L, L)] * N_OUT
            for j in range(N_OUT):
                a = plsc.load_gather(y_v, [rv + j])
                b = plsc.load_gather(y_v, [cv + j])
                e2 = jnp.exp((a + b) * 2.0)
                t = 1.0 - 2.0 / (e2 + 1.0)
                outs[j][pl.ds(g * L, L)] = t
            return carry

        lax.fori_loop(0, GROUPS, body, 0)
        for j in range(N_OUT):
            pltpu.sync_copy(outs[j], out_hbm.at[j, pl.ds(base, EPW)])

    return edge_kernel


def kernel(x, edge_index, W):
    y = _node_proj(x, W.T)
    out = _make_edge_kernel()(y.reshape(-1), edge_index[0], edge_index[1])
    return out.T.reshape(N_EDGES, 2, 2)


# speedup vs baseline: 46.8074x; 1.6718x over previous
import functools

import jax
import jax.numpy as jnp
from jax import lax
from jax.experimental import pallas as pl
from jax.experimental.pallas import tpu as pltpu
from jax.experimental.pallas import tpu_sc as plsc

N_NODES = 10000
N_EDGES = 320000
D_FEAT = 128
N_OUT = 4

NC, NS, L = 2, 16, 16
NW = NC * NS
EPW = N_EDGES // NW
GROUPS = EPW // L


def _mm_body(x_ref, wt_ref, y_ref):
    y_ref[...] = jnp.dot(x_ref[...], wt_ref[...],
                         preferred_element_type=jnp.float32)


def _node_proj(x, wt):
    return pl.pallas_call(
        _mm_body,
        out_shape=jax.ShapeDtypeStruct((N_NODES, N_OUT), jnp.float32),
    )(x, wt)


@functools.cache
def _make_edge_kernel():
    mesh = plsc.VectorSubcoreMesh(core_axis_name="c", subcore_axis_name="s")

    @functools.partial(
        pl.kernel,
        mesh=mesh,
        out_type=jax.ShapeDtypeStruct((N_OUT, N_EDGES), jnp.float32),
        scratch_types=[
            pltpu.VMEM((N_NODES * N_OUT,), jnp.float32),
            pltpu.VMEM((EPW,), jnp.int32),
            pltpu.VMEM((EPW,), jnp.int32),
        ] + [pltpu.VMEM((EPW,), jnp.float32) for _ in range(N_OUT)],
        compiler_params=pltpu.CompilerParams(
            needs_layout_passes=False, use_tc_tiling_on_sc=False),
    )
    def edge_kernel(y_hbm, row_hbm, col_hbm, out_hbm,
                    y_v, rows_v, cols_v, o0, o1, o2, o3):
        outs = (o0, o1, o2, o3)
        wid = lax.axis_index("s") * NC + lax.axis_index("c")
        base = wid * EPW
        pltpu.sync_copy(y_hbm, y_v)
        pltpu.sync_copy(row_hbm.at[pl.ds(base, EPW)], rows_v)
        pltpu.sync_copy(col_hbm.at[pl.ds(base, EPW)], cols_v)

        @plsc.parallel_loop(0, GROUPS, unroll=8)
        def body(g):
            rv = rows_v[pl.ds(g * L, L)] * N_OUT
            cv = cols_v[pl.ds(g * L, L)] * N_OUT
            for j in range(N_OUT):
                a = plsc.load_gather(y_v, [rv + j])
                b = plsc.load_gather(y_v, [cv + j])
                e2 = jnp.exp((a + b) * 2.0)
                t = 1.0 - 2.0 / (e2 + 1.0)
                outs[j][pl.ds(g * L, L)] = t
        for j in range(N_OUT):
            pltpu.sync_copy(outs[j], out_hbm.at[j, pl.ds(base, EPW)])

    return edge_kernel


def kernel(x, edge_index, W):
    y = _node_proj(x, W.T)
    out = _make_edge_kernel()(y.reshape(-1), edge_index[0], edge_index[1])
    return out.T.reshape(N_EDGES, 2, 2)


# speedup vs baseline: 56.6138x; 1.2095x over previous
import functools

import jax
import jax.numpy as jnp
from jax import lax
from jax.experimental import pallas as pl
from jax.experimental.pallas import tpu as pltpu
from jax.experimental.pallas import tpu_sc as plsc

N_NODES = 10000
N_EDGES = 320000
D_FEAT = 128
N_OUT = 4

NC, NS, L = 2, 16, 16
NW = NC * NS
EPW = N_EDGES // NW
GROUPS = EPW // L


def _mm_body(w_ref, x_ref, yt_ref):
    yt_ref[...] = jax.lax.dot_general(
        w_ref[...], x_ref[...], (((1,), (1,)), ((), ())),
        preferred_element_type=jnp.float32)


def _node_proj_t(w, x):
    return pl.pallas_call(
        _mm_body,
        out_shape=jax.ShapeDtypeStruct((N_OUT, N_NODES), jnp.float32),
    )(w, x)


@functools.cache
def _make_edge_kernel():
    mesh = plsc.VectorSubcoreMesh(core_axis_name="c", subcore_axis_name="s")

    @functools.partial(
        pl.kernel,
        mesh=mesh,
        out_type=jax.ShapeDtypeStruct((N_OUT, N_EDGES), jnp.float32),
        scratch_types=[
            pltpu.VMEM((N_NODES * N_OUT,), jnp.float32),
            pltpu.VMEM((EPW,), jnp.int32),
            pltpu.VMEM((EPW,), jnp.int32),
        ] + [pltpu.VMEM((EPW,), jnp.float32) for _ in range(N_OUT)],
        compiler_params=pltpu.CompilerParams(
            needs_layout_passes=False, use_tc_tiling_on_sc=False),
    )
    def edge_kernel(y_hbm, row_hbm, col_hbm, out_hbm,
                    y_v, rows_v, cols_v, o0, o1, o2, o3):
        outs = (o0, o1, o2, o3)
        wid = lax.axis_index("s") * NC + lax.axis_index("c")
        base = wid * EPW
        pltpu.sync_copy(y_hbm, y_v)
        pltpu.sync_copy(row_hbm.at[pl.ds(base, EPW)], rows_v)
        pltpu.sync_copy(col_hbm.at[pl.ds(base, EPW)], cols_v)

        @plsc.parallel_loop(0, GROUPS, unroll=8)
        def body(g):
            rv = rows_v[pl.ds(g * L, L)]
            cv = cols_v[pl.ds(g * L, L)]
            for j in range(N_OUT):
                a = plsc.load_gather(y_v, [rv + (j * N_NODES)])
                b = plsc.load_gather(y_v, [cv + (j * N_NODES)])
                e2 = jnp.exp((a + b) * 2.0)
                t = 1.0 - 2.0 / (e2 + 1.0)
                outs[j][pl.ds(g * L, L)] = t
        for j in range(N_OUT):
            pltpu.sync_copy(outs[j], out_hbm.at[j, pl.ds(base, EPW)])

    return edge_kernel


def kernel(x, edge_index, W):
    yt = _node_proj_t(W, x)
    out = _make_edge_kernel()(yt.reshape(-1), edge_index[0], edge_index[1])
    return out.T.reshape(N_EDGES, 2, 2)


# speedup vs baseline: 72.2458x; 1.2761x over previous
import functools

import jax
import jax.numpy as jnp
from jax import lax
from jax.experimental import pallas as pl
from jax.experimental.pallas import tpu as pltpu
from jax.experimental.pallas import tpu_sc as plsc

N_NODES = 10000
N_EDGES = 320000
D_FEAT = 128
N_OUT = 4

NC, NS, L = 2, 16, 16
NW = NC * NS
BLK = 128
N_BLKS = N_EDGES // BLK
BASE_BPW = N_BLKS // NW
EXTRA = N_BLKS - BASE_BPW * NW
MAX_BPW = BASE_BPW + 1
GPB = BLK // L


def _mm_body(w_ref, x_ref, yt_ref):
    yt_ref[...] = jax.lax.dot_general(
        w_ref[...], x_ref[...], (((1,), (1,)), ((), ())),
        preferred_element_type=jnp.float32)


def _node_proj_t(w, x):
    return pl.pallas_call(
        _mm_body,
        out_shape=jax.ShapeDtypeStruct((N_OUT, N_NODES), jnp.float32),
    )(w, x)


@functools.cache
def _make_edge_kernel():
    mesh = plsc.VectorSubcoreMesh(core_axis_name="c", subcore_axis_name="s")

    @functools.partial(
        pl.kernel,
        mesh=mesh,
        out_type=jax.ShapeDtypeStruct((N_BLKS, N_OUT, BLK), jnp.float32),
        scratch_types=[
            pltpu.VMEM((N_NODES * N_OUT,), jnp.float32),
            pltpu.VMEM((MAX_BPW, 2, BLK), jnp.int32),
            pltpu.VMEM((MAX_BPW, N_OUT, BLK), jnp.float32),
        ],
        compiler_params=pltpu.CompilerParams(
            needs_layout_passes=False, use_tc_tiling_on_sc=False),
    )
    def edge_kernel(y_hbm, ei_hbm, out_hbm, y_v, ei_v, out_v):
        wid = lax.axis_index("s") * NC + lax.axis_index("c")
        nb = jnp.where(wid < EXTRA, MAX_BPW, BASE_BPW)
        b0 = wid * BASE_BPW + jnp.minimum(wid, EXTRA)
        pltpu.sync_copy(y_hbm, y_v)
        pltpu.sync_copy(ei_hbm.at[pl.ds(b0, nb)], ei_v.at[pl.ds(0, nb)])

        @plsc.parallel_loop(0, nb * GPB, unroll=8)
        def body(g):
            blk = g // GPB
            el0 = (g % GPB) * L
            rv = ei_v[blk, 0, pl.ds(el0, L)]
            cv = ei_v[blk, 1, pl.ds(el0, L)]
            for j in range(N_OUT):
                a = plsc.load_gather(y_v, [rv + (j * N_NODES)])
                b = plsc.load_gather(y_v, [cv + (j * N_NODES)])
                e2 = jnp.exp((a + b) * 2.0)
                t = 1.0 - 2.0 / (e2 + 1.0)
                out_v[blk, j, pl.ds(el0, L)] = t

        pltpu.sync_copy(out_v.at[pl.ds(0, nb)], out_hbm.at[pl.ds(b0, nb)])

    return edge_kernel


def kernel(x, edge_index, W):
    yt = _node_proj_t(W, x)
    ei_blocks = edge_index.reshape(2, N_BLKS, BLK).transpose(1, 0, 2)
    out = _make_edge_kernel()(yt.reshape(-1), ei_blocks)
    return out.transpose(0, 2, 1).reshape(N_EDGES, N_OUT).reshape(
        N_EDGES, 2, 2)


# speedup vs baseline: 78.9151x; 1.0923x over previous
import functools

import jax
import jax.numpy as jnp
from jax import lax
from jax.experimental import pallas as pl
from jax.experimental.pallas import tpu as pltpu
from jax.experimental.pallas import tpu_sc as plsc

N_NODES = 10000
N_EDGES = 320000
D_FEAT = 128
N_OUT = 4

NC, NS, L = 2, 16, 16
NW = NC * NS
BLK = 128
N_BLKS = N_EDGES // BLK
BASE_BPW = N_BLKS // NW
EXTRA = N_BLKS - BASE_BPW * NW
MAX_BPW = BASE_BPW + 1
GPB = BLK // L


def _mm_body(w_ref, x_ref, yt_ref):
    yt_ref[...] = jax.lax.dot_general(
        w_ref[...], x_ref[...], (((1,), (1,)), ((), ())),
        preferred_element_type=jnp.float32)


def _node_proj_t(w, x):
    return pl.pallas_call(
        _mm_body,
        out_shape=jax.ShapeDtypeStruct((N_OUT, N_NODES), jnp.float32),
    )(w, x)


@functools.cache
def _make_edge_kernel():
    mesh = plsc.VectorSubcoreMesh(core_axis_name="c", subcore_axis_name="s")

    @functools.partial(
        pl.kernel,
        mesh=mesh,
        out_type=jax.ShapeDtypeStruct((2, N_BLKS, 2, BLK), jnp.float32),
        scratch_types=[
            pltpu.VMEM((N_NODES * N_OUT,), jnp.float32),
            pltpu.VMEM((MAX_BPW, 2, BLK), jnp.int32),
            pltpu.VMEM((2, MAX_BPW, 2, BLK), jnp.float32),
        ],
        compiler_params=pltpu.CompilerParams(
            needs_layout_passes=False, use_tc_tiling_on_sc=False),
    )
    def edge_kernel(y_hbm, ei_hbm, out_hbm, y_v, ei_v, out_v):
        wid = lax.axis_index("s") * NC + lax.axis_index("c")
        nb = jnp.where(wid < EXTRA, MAX_BPW, BASE_BPW)
        b0 = wid * BASE_BPW + jnp.minimum(wid, EXTRA)
        pltpu.sync_copy(y_hbm, y_v)
        pltpu.sync_copy(ei_hbm.at[pl.ds(b0, nb)], ei_v.at[pl.ds(0, nb)])

        @plsc.parallel_loop(0, nb * GPB, unroll=8)
        def body(g):
            blk = g // GPB
            el0 = (g % GPB) * L
            rv = ei_v[blk, 0, pl.ds(el0, L)]
            cv = ei_v[blk, 1, pl.ds(el0, L)]
            for j in range(N_OUT):
                a = plsc.load_gather(y_v, [rv + (j * N_NODES)])
                b = plsc.load_gather(y_v, [cv + (j * N_NODES)])
                e2 = jnp.exp((a + b) * 2.0)
                t = 1.0 - 2.0 / (e2 + 1.0)
                out_v[j // 2, blk, j % 2, pl.ds(el0, L)] = t

        for i in range(2):
            for jj in range(2):
                pltpu.sync_copy(out_v.at[i, pl.ds(0, nb), jj],
                                out_hbm.at[i, pl.ds(b0, nb), jj])

    return edge_kernel


def kernel(x, edge_index, W):
    yt = _node_proj_t(W, x)
    ei_blocks = edge_index.reshape(2, N_BLKS, BLK).transpose(1, 0, 2)
    out = _make_edge_kernel()(yt.reshape(-1), ei_blocks)
    return out.transpose(1, 3, 0, 2).reshape(N_EDGES, 2, 2)
